# baseline (speedup 1.0000x reference)
import functools

import numpy as np
import jax
import jax.numpy as jnp
from jax import lax
from jax.experimental import pallas as pl
from jax.experimental.pallas import tpu as pltpu
from jax.experimental.pallas import tpu_sc as plsc

N_NODES = 10000
N_EDGES = 320000
N_GRAPHS = 8
NBASIS = 10

_BN = 2000
_BE = 2000
_NC = 2
_NS = 16
_K = 80
_NSLOT = 3


def _embed_body(x_ref, z_ref, wx_ref, bx_ref, wz_ref, bz_ref, h_ref, z0_ref):
    xb = jnp.maximum(x_ref[...], 0.0)
    h = jnp.dot(xb, wx_ref[...], preferred_element_type=jnp.float32) + bx_ref[...]
    h_ref[...] = jnp.maximum(h, 0.0)
    zb = jnp.maximum(z_ref[...], 0.0)
    z0 = jnp.dot(zb, wz_ref[...], preferred_element_type=jnp.float32) + bz_ref[...]
    z0_ref[...] = jnp.maximum(z0, 0.0)


def _embed(x, z, Wemx, bemx, Wemz, bemz):
    grid = (N_NODES // _BN,)
    full = lambda shape: pl.BlockSpec(shape, lambda i: (0, 0))
    return pl.pallas_call(
        _embed_body,
        grid=grid,
        in_specs=[
            pl.BlockSpec((_BN, 128), lambda i: (i, 0)),
            pl.BlockSpec((_BN, 128), lambda i: (i, 0)),
            full((128, 64)), full((1, 64)), full((128, 32)), full((1, 32)),
        ],
        out_specs=[
            pl.BlockSpec((_BN, 64), lambda i: (i, 0)),
            pl.BlockSpec((_BN, 32), lambda i: (i, 0)),
        ],
        out_shape=[
            jax.ShapeDtypeStruct((N_NODES, 64), jnp.float32),
            jax.ShapeDtypeStruct((N_NODES, 32), jnp.float32),
        ],
    )(x, z, Wemx, bemx.reshape(1, 64), Wemz, bemz.reshape(1, 32))


def _radial_body(el_ref, ev_ref, rmax_ref, wra_ref, bra_ref, wrb_ref, wsh_ref,
                 out_ref):
    el = el_ref[...]
    step = rmax_ref[0, 0] / (NBASIS - 1)
    j = lax.broadcasted_iota(jnp.int32, (1, NBASIS), 1).astype(jnp.float32)
    diff = (el - j * step) / step
    eemb = jnp.exp(-diff * diff) * (1.0 / 1.12)

    ev = ev_ref[...]
    n = jnp.sqrt(jnp.sum(ev * ev, axis=1, keepdims=True)) + 1e-12
    u = (ev / n) * jnp.sqrt(3.0)
    ones = jnp.ones_like(el)
    sh = jnp.concatenate([ones, u[:, 1:2], u[:, 2:3], u[:, 0:1]], axis=1)

    a = jnp.dot(eemb, wra_ref[...], preferred_element_type=jnp.float32) + bra_ref[...]
    a = a * jax.nn.sigmoid(a)
    rad = jnp.dot(a, wrb_ref[...], preferred_element_type=jnp.float32)
    gsc = jnp.dot(sh, wsh_ref[...], preferred_element_type=jnp.float32)
    rad = rad * gsc
    cr = rad.shape[1] // 2
    out_ref[0] = rad[:, :cr]
    out_ref[1] = rad[:, cr:]


def _radial(edge_len, edge_vec, r_max, Wra, bra, Wrb2, wsh):
    cr = Wrb2.shape[1] // 2
    grid = (N_EDGES // _BE,)
    full = lambda shape: pl.BlockSpec(shape, lambda i: tuple(0 for _ in shape))
    return pl.pallas_call(
        _radial_body,
        grid=grid,
        in_specs=[
            pl.BlockSpec((_BE, 1), lambda i: (i, 0)),
            pl.BlockSpec((_BE, 3), lambda i: (i, 0)),
            full((1, 1)),
            full((NBASIS, 64)), full((1, 64)), full((64, 2 * cr)), full((4, 1)),
        ],
        out_specs=pl.BlockSpec((_NC, _BE, cr), lambda i: (0, i, 0)),
        out_shape=jax.ShapeDtypeStruct((_NC, N_EDGES, cr), jnp.float32),
    )(edge_len.reshape(N_EDGES, 1), edge_vec, r_max.reshape(1, 1),
      Wra, bra.reshape(1, 64), Wrb2, wsh.reshape(4, 1))


def _pre_body(h_ref, z0_ref, ws_ref, wz_ref, wm_ref, sc_ref, hm_ref):
    h = h_ref[...]
    sc_ref[...] = (jnp.dot(h, ws_ref[...], preferred_element_type=jnp.float32)
                   + jnp.dot(z0_ref[...], wz_ref[...], preferred_element_type=jnp.float32))
    hm = jnp.dot(h, wm_ref[...], preferred_element_type=jnp.float32)
    ch = hm.shape[1] // 2
    hm_ref[0] = hm[:, :ch]
    hm_ref[1] = hm[:, ch:]


def _pre(h, z0, Ws, Wz, Wm):
    cin, cout = Ws.shape
    ch = cout // 2
    grid = (N_NODES // _BN,)
    full = lambda shape: pl.BlockSpec(shape, lambda i: (0, 0))
    return pl.pallas_call(
        _pre_body,
        grid=grid,
        in_specs=[
            pl.BlockSpec((_BN, cin), lambda i: (i, 0)),
            pl.BlockSpec((_BN, 32), lambda i: (i, 0)),
            full((cin, cout)), full((32, cout)), full((cin, cout)),
        ],
        out_specs=[
            pl.BlockSpec((_BN, cout), lambda i: (i, 0)),
            pl.BlockSpec((_NC, _BN, ch), lambda i: (0, i, 0)),
        ],
        out_shape=[
            jax.ShapeDtypeStruct((N_NODES, cout), jnp.float32),
            jax.ShapeDtypeStruct((_NC, N_NODES, ch), jnp.float32),
        ],
    )(h, z0, Ws, Wz, Wm)


def _gate_body(sc_ref, agga_ref, aggb_ref, deg_ref, h_ref):
    agg = jnp.concatenate([agga_ref[0], aggb_ref[0]], axis=1)
    tot = sc_ref[...] + agg / deg_ref[...]
    s = tot[:, :32]
    s = s * jax.nn.sigmoid(s)
    g = jax.nn.sigmoid(tot[:, 32:64])
    row = lax.broadcasted_iota(jnp.int32, (32, 96), 0)
    col = lax.broadcasted_iota(jnp.int32, (32, 96), 1)
    rep = (col // 3 == row).astype(jnp.float32)
    g3 = jnp.dot(g, rep, preferred_element_type=jnp.float32)
    h_ref[...] = jnp.concatenate([s, tot[:, 64:160] * g3], axis=1)


def _gate(sc, agg, node_deg):
    grid = (N_NODES // _BN,)
    return pl.pallas_call(
        _gate_body,
        grid=grid,
        in_specs=[
            pl.BlockSpec((_BN, 160), lambda i: (i, 0)),
            pl.BlockSpec((1, _BN, 80), lambda i: (0, i, 0)),
            pl.BlockSpec((1, _BN, 80), lambda i: (1, i, 0)),
            pl.BlockSpec((_BN, 1), lambda i: (i, 0)),
        ],
        out_specs=pl.BlockSpec((_BN, 128), lambda i: (i, 0)),
        out_shape=jax.ShapeDtypeStruct((N_NODES, 128), jnp.float32),
    )(sc, agg, agg, node_deg)


def _final_body(sc_ref, agga_ref, aggb_ref, deg_ref, batch_ref, wc_ref, bc_ref,
                out_ref, pooled_acc, cnt_acc):
    i = pl.program_id(0)

    @pl.when(i == 0)
    def _():
        pooled_acc[...] = jnp.zeros_like(pooled_acc)
        cnt_acc[...] = jnp.zeros_like(cnt_acc)

    agg = jnp.concatenate([agga_ref[0], aggb_ref[0]], axis=1)
    h = sc_ref[...] + agg / deg_ref[...]
    b = batch_ref[...]
    rows = []
    cnts = []
    for g in range(N_GRAPHS):
        m = (b == g).astype(jnp.float32)
        cnts.append(jnp.sum(m, axis=0, keepdims=True))
        rows.append(jnp.sum(h * m, axis=0, keepdims=True))
    pooled_acc[...] += jnp.concatenate(rows, axis=0)
    cnt_acc[...] += jnp.concatenate(cnts, axis=0)

    @pl.when(i == pl.num_programs(0) - 1)
    def _():
        pooled = pooled_acc[...] / jnp.maximum(cnt_acc[...], 1.0)
        logit = (jnp.dot(pooled, wc_ref[...], preferred_element_type=jnp.float32)
                 + bc_ref[...])
        out_ref[...] = jax.nn.sigmoid(logit)


def _final(sc, agg, node_deg, batch, Wc, bc):
    full = lambda shape: pl.BlockSpec(shape, lambda i: tuple(0 for _ in shape))
    return pl.pallas_call(
        _final_body,
        grid=(N_NODES // _BN,),
        in_specs=[
            pl.BlockSpec((_BN, 64), lambda i: (i, 0)),
            pl.BlockSpec((1, _BN, 32), lambda i: (0, i, 0)),
            pl.BlockSpec((1, _BN, 32), lambda i: (1, i, 0)),
            pl.BlockSpec((_BN, 1), lambda i: (i, 0)),
            pl.BlockSpec((_BN, 1), lambda i: (i, 0)),
            full((64, 1)),
            full((1, 1)),
        ],
        out_specs=full((N_GRAPHS, 1)),
        out_shape=jax.ShapeDtypeStruct((N_GRAPHS, 1), jnp.float32),
        scratch_shapes=[
            pltpu.VMEM((N_GRAPHS, 64), jnp.float32),
            pltpu.VMEM((N_GRAPHS, 1), jnp.float32),
        ],
    )(sc, agg, agg, node_deg, batch.reshape(N_NODES, 1), Wc, bc.reshape(1, 1))


def _sc_edge(hm3, src, dst, rad3, C2, CR):
    e_per_t = N_EDGES // _NS
    n_chunks = e_per_t // _K
    rpt = 624
    tail_base = rpt * _NS
    tail_rows = N_NODES - tail_base
    mesh = plsc.VectorSubcoreMesh(core_axis_name="c", subcore_axis_name="s")

    @functools.partial(
        pl.kernel,
        mesh=mesh,
        out_type=jax.ShapeDtypeStruct((_NC, N_NODES, C2), jnp.float32),
        scratch_types=[
            pltpu.VMEM_SHARED((N_NODES, C2), jnp.float32),
            pltpu.VMEM((_NSLOT, _K), jnp.int32),
            pltpu.VMEM((_NSLOT, _K), jnp.int32),
            pltpu.VMEM((_NSLOT, _K, CR), jnp.float32),
            pltpu.VMEM((_NSLOT, _K, C2), jnp.float32),
        ] + [pltpu.SemaphoreType.DMA] * (3 * _NSLOT),
        compiler_params=pltpu.CompilerParams(use_tc_tiling_on_sc=False),
    )
    def k(hm_hbm, src_hbm, dst_hbm, rad_hbm, out_hbm, agg_sh, src_v, dst_v,
          rad_v, rows_v, *sems):
        sem_in = sems[0:_NSLOT]
        sem_g = sems[_NSLOT:2 * _NSLOT]
        sem_s = sems[2 * _NSLOT:3 * _NSLOT]
        cid = lax.axis_index("c")
        sid = lax.axis_index("s")

        zero = jnp.zeros((16,), jnp.float32)

        def zrow(i, _):
            for c in range(C2 // 16):
                rows_v[0, i, pl.ds(c * 16, 16)] = zero
            return 0

        lax.fori_loop(0, 16, zrow, 0)

        def zcopy(i, _):
            pltpu.sync_copy(rows_v.at[0, pl.ds(0, 16)],
                            agg_sh.at[pl.ds(sid * rpt + i * 16, 16)])
            return 0

        lax.fori_loop(0, rpt // 16, zcopy, 0)

        @pl.when(sid == _NS - 1)
        def _():
            pltpu.sync_copy(rows_v.at[0, pl.ds(0, 16)],
                            agg_sh.at[pl.ds(tail_base, tail_rows)])

        plsc.subcore_barrier()

        def in_start(i, b):
            base = sid * e_per_t + i * _K
            pltpu.async_copy(src_hbm.at[pl.ds(base, _K)], src_v.at[b], sem_in[b])
            pltpu.async_copy(dst_hbm.at[pl.ds(base, _K)], dst_v.at[b], sem_in[b])
            pltpu.async_copy(rad_hbm.at[cid, pl.ds(base, _K)], rad_v.at[b],
                             sem_in[b])

        def in_wait(i, b):
            base = sid * e_per_t + i * _K
            pltpu.make_async_copy(src_hbm.at[pl.ds(base, _K)], src_v.at[b],
                                  sem_in[b]).wait()
            pltpu.make_async_copy(dst_hbm.at[pl.ds(base, _K)], dst_v.at[b],
                                  sem_in[b]).wait()
            pltpu.make_async_copy(rad_hbm.at[cid, pl.ds(base, _K)],
                                  rad_v.at[b], sem_in[b]).wait()

        def gather_start(b):
            pltpu.async_copy(hm_hbm.at[cid].at[src_v.at[b]], rows_v.at[b],
                             sem_g[b])

        def gather_wait(b):
            pltpu.make_async_copy(hm_hbm.at[cid].at[src_v.at[b]], rows_v.at[b],
                                  sem_g[b]).wait()

        def scatter_start(b):
            pltpu.async_copy(rows_v.at[b], agg_sh.at[dst_v.at[b]], sem_s[b],
                             add=True)

        def scatter_wait(b):
            pltpu.make_async_copy(rows_v.at[b], agg_sh.at[dst_v.at[b]],
                                  sem_s[b]).wait()

        def mul(b):
            def mrow(j, _):
                for c in range(C2 // 16):
                    sl = pl.ds(c * 16, 16)
                    rows_v[b, j, sl] = rows_v[b, j, sl] * rad_v[b, j, sl]
                return 0

            lax.fori_loop(0, _K, mrow, 0)

        in_start(0, 0)
        in_start(1, 1)
        in_wait(0, 0)
        gather_start(0)

        def body(i, _):
            for b in range(_NSLOT):
                @pl.when(i % _NSLOT == b)
                def _():
                    nb = (b + 1) % _NSLOT
                    pb = (b + 2) % _NSLOT

                    @pl.when(i + 1 < n_chunks)
                    def _():
                        in_wait(i + 1, nb)
                        gather_start(nb)

                    gather_wait(b)
                    mul(b)
                    scatter_start(b)

                    @pl.when(i + 2 < n_chunks)
                    def _():
                        @pl.when(i >= 1)
                        def _():
                            scatter_wait(pb)
                        in_start(i + 2, pb)
            return 0

        lax.fori_loop(0, n_chunks, body, 0)

        for j in range(_NSLOT):
            i = n_chunks - _NSLOT + j
            scatter_wait(i % _NSLOT)

        plsc.subcore_barrier()
        pltpu.sync_copy(agg_sh.at[pl.ds(sid * rpt, rpt)],
                        out_hbm.at[cid, pl.ds(sid * rpt, rpt)])

        @pl.when(sid == _NS - 1)
        def _():
            pltpu.sync_copy(agg_sh.at[pl.ds(tail_base, tail_rows)],
                            out_hbm.at[cid, pl.ds(tail_base, tail_rows)])

    return k(hm3, src, dst, rad3)


def kernel(x, z, edge_index, edge_vec, edge_len, r_max, node_deg, numb, batch,
           Wemx, bemx, Wemz, bemz,
           Ws1, Wz1, Wm1, Wra1, bra1, Wrb1, wsh1,
           Ws2, Wz2, Wm2, Wra2, bra2, Wrb2, wsh2,
           Ws3, Wz3, Wm3, Wra3, bra3, Wrb3, wsh3,
           Wc, bc):
    src = edge_index[0]
    dst = edge_index[1]

    h, z0 = _embed(x, z, Wemx, bemx, Wemz, bemz)

    rad1 = _radial(edge_len, edge_vec, r_max, Wra1, bra1, Wrb1, wsh1)
    rad2 = _radial(edge_len, edge_vec, r_max, Wra2, bra2, Wrb2, wsh2)
    rad3 = _radial(edge_len, edge_vec, r_max, Wra3, bra3, Wrb3, wsh3)

    sc1, hm1 = _pre(h, z0, Ws1, Wz1, Wm1)
    agg1 = _sc_edge(hm1, src, dst, rad1, 80, 80)
    h = _gate(sc1, agg1, node_deg)

    sc2, hm2 = _pre(h, z0, Ws2, Wz2, Wm2)
    agg2 = _sc_edge(hm2, src, dst, rad2, 80, 80)
    h = _gate(sc2, agg2, node_deg)

    sc3, hm3 = _pre(h, z0, Ws3, Wz3, Wm3)
    agg3 = _sc_edge(hm3, src, dst, rad3, 32, 32)
    return _final(sc3, agg3, node_deg, batch, Wc, bc)


# speedup vs baseline: 1.1210x; 1.1210x over previous
import functools

import numpy as np
import jax
import jax.numpy as jnp
from jax import lax
from jax.experimental import pallas as pl
from jax.experimental.pallas import tpu as pltpu
from jax.experimental.pallas import tpu_sc as plsc

N_NODES = 10000
N_EDGES = 320000
N_GRAPHS = 8
NBASIS = 10

_BN = 2000
_BE = 2000
_NC = 2
_NS = 16
_K = 80
_NSLOT = 3


def _embed_body(x_ref, z_ref, wx_ref, bx_ref, wz_ref, bz_ref, h_ref, z0_ref):
    xb = jnp.maximum(x_ref[...], 0.0)
    h = jnp.dot(xb, wx_ref[...], preferred_element_type=jnp.float32) + bx_ref[...]
    h_ref[...] = jnp.maximum(h, 0.0)
    zb = jnp.maximum(z_ref[...], 0.0)
    z0 = jnp.dot(zb, wz_ref[...], preferred_element_type=jnp.float32) + bz_ref[...]
    z0_ref[...] = jnp.maximum(z0, 0.0)


def _embed(x, z, Wemx, bemx, Wemz, bemz):
    grid = (N_NODES // _BN,)
    full = lambda shape: pl.BlockSpec(shape, lambda i: (0, 0))
    return pl.pallas_call(
        _embed_body,
        grid=grid,
        in_specs=[
            pl.BlockSpec((_BN, 128), lambda i: (i, 0)),
            pl.BlockSpec((_BN, 128), lambda i: (i, 0)),
            full((128, 64)), full((1, 64)), full((128, 32)), full((1, 32)),
        ],
        out_specs=[
            pl.BlockSpec((_BN, 64), lambda i: (i, 0)),
            pl.BlockSpec((_BN, 32), lambda i: (i, 0)),
        ],
        out_shape=[
            jax.ShapeDtypeStruct((N_NODES, 64), jnp.float32),
            jax.ShapeDtypeStruct((N_NODES, 32), jnp.float32),
        ],
    )(x, z, Wemx, bemx.reshape(1, 64), Wemz, bemz.reshape(1, 32))


def _radial_body(el_ref, ev_ref, rmax_ref,
                 wra1_ref, bra1_ref, wrb1_ref, wsh1_ref,
                 wra2_ref, bra2_ref, wrb2_ref, wsh2_ref,
                 wra3_ref, bra3_ref, wrb3_ref, wsh3_ref,
                 r1_ref, r2_ref, r3_ref):
    el = el_ref[...]
    step = rmax_ref[0, 0] / (NBASIS - 1)
    j = lax.broadcasted_iota(jnp.int32, (1, NBASIS), 1).astype(jnp.float32)
    diff = (el - j * step) / step
    eemb = jnp.exp(-diff * diff) * (1.0 / 1.12)

    ev = ev_ref[...]
    n = jnp.sqrt(jnp.sum(ev * ev, axis=1, keepdims=True)) + 1e-12
    u = (ev / n) * jnp.sqrt(3.0)
    ones = jnp.ones_like(el)
    sh = jnp.concatenate([ones, u[:, 1:2], u[:, 2:3], u[:, 0:1]], axis=1)

    for wra, bra, wrb, wsh, out_ref in (
        (wra1_ref, bra1_ref, wrb1_ref, wsh1_ref, r1_ref),
        (wra2_ref, bra2_ref, wrb2_ref, wsh2_ref, r2_ref),
        (wra3_ref, bra3_ref, wrb3_ref, wsh3_ref, r3_ref),
    ):
        a = jnp.dot(eemb, wra[...], preferred_element_type=jnp.float32) + bra[...]
        a = a * jax.nn.sigmoid(a)
        rad = jnp.dot(a, wrb[...], preferred_element_type=jnp.float32)
        gsc = jnp.dot(sh, wsh[...], preferred_element_type=jnp.float32)
        rad = rad * gsc
        ch = rad.shape[1] // 2
        out_ref[0] = rad[:, :ch]
        out_ref[1] = rad[:, ch:]


def _radials(edge_len, edge_vec, r_max,
             Wra1, bra1, Wrb1, wsh1, Wra2, bra2, Wrb2, wsh2, Wra3, bra3, Wrb3, wsh3):
    grid = (N_EDGES // _BE,)
    full = lambda shape: pl.BlockSpec(shape, lambda i: tuple(0 for _ in shape))
    ws = []
    for wra, bra, wrb, wsh in ((Wra1, bra1, Wrb1, wsh1), (Wra2, bra2, Wrb2, wsh2),
                               (Wra3, bra3, Wrb3, wsh3)):
        ws += [wra, bra.reshape(1, 64), wrb, wsh.reshape(4, 1)]
    in_specs = [
        pl.BlockSpec((_BE, 1), lambda i: (i, 0)),
        pl.BlockSpec((_BE, 3), lambda i: (i, 0)),
        full((1, 1)),
    ]
    for k in range(3):
        cout = 64 if k == 2 else 160
        in_specs += [full((NBASIS, 64)), full((1, 64)), full((64, cout)), full((4, 1))]
    out_specs = [
        pl.BlockSpec((_NC, _BE, 80), lambda i: (0, i, 0)),
        pl.BlockSpec((_NC, _BE, 80), lambda i: (0, i, 0)),
        pl.BlockSpec((_NC, _BE, 32), lambda i: (0, i, 0)),
    ]
    return pl.pallas_call(
        _radial_body,
        grid=grid,
        in_specs=in_specs,
        out_specs=out_specs,
        out_shape=[
            jax.ShapeDtypeStruct((_NC, N_EDGES, 80), jnp.float32),
            jax.ShapeDtypeStruct((_NC, N_EDGES, 80), jnp.float32),
            jax.ShapeDtypeStruct((_NC, N_EDGES, 32), jnp.float32),
        ],
    )(edge_len.reshape(N_EDGES, 1), edge_vec, r_max.reshape(1, 1), *ws)


def _pre_body(h_ref, z0_ref, ws_ref, wz_ref, wm_ref, sc_ref, hm_ref):
    h = h_ref[...]
    sc_ref[...] = (jnp.dot(h, ws_ref[...], preferred_element_type=jnp.float32)
                   + jnp.dot(z0_ref[...], wz_ref[...], preferred_element_type=jnp.float32))
    hm = jnp.dot(h, wm_ref[...], preferred_element_type=jnp.float32)
    ch = hm.shape[1] // 2
    hm_ref[0] = hm[:, :ch]
    hm_ref[1] = hm[:, ch:]


def _pre(h, z0, Ws, Wz, Wm):
    cin, cout = Ws.shape
    ch = cout // 2
    grid = (N_NODES // _BN,)
    full = lambda shape: pl.BlockSpec(shape, lambda i: (0, 0))
    return pl.pallas_call(
        _pre_body,
        grid=grid,
        in_specs=[
            pl.BlockSpec((_BN, cin), lambda i: (i, 0)),
            pl.BlockSpec((_BN, 32), lambda i: (i, 0)),
            full((cin, cout)), full((32, cout)), full((cin, cout)),
        ],
        out_specs=[
            pl.BlockSpec((_BN, cout), lambda i: (i, 0)),
            pl.BlockSpec((_NC, _BN, ch), lambda i: (0, i, 0)),
        ],
        out_shape=[
            jax.ShapeDtypeStruct((N_NODES, cout), jnp.float32),
            jax.ShapeDtypeStruct((_NC, N_NODES, ch), jnp.float32),
        ],
    )(h, z0, Ws, Wz, Wm)


def _gate_body(sc_ref, agga_ref, aggb_ref, deg_ref, h_ref):
    agg = jnp.concatenate([agga_ref[0], aggb_ref[0]], axis=1)
    tot = sc_ref[...] + agg / deg_ref[...]
    s = tot[:, :32]
    s = s * jax.nn.sigmoid(s)
    g = jax.nn.sigmoid(tot[:, 32:64])
    row = lax.broadcasted_iota(jnp.int32, (32, 96), 0)
    col = lax.broadcasted_iota(jnp.int32, (32, 96), 1)
    rep = (col // 3 == row).astype(jnp.float32)
    g3 = jnp.dot(g, rep, preferred_element_type=jnp.float32)
    h_ref[...] = jnp.concatenate([s, tot[:, 64:160] * g3], axis=1)


def _gate(sc, agg, node_deg):
    grid = (N_NODES // _BN,)
    return pl.pallas_call(
        _gate_body,
        grid=grid,
        in_specs=[
            pl.BlockSpec((_BN, 160), lambda i: (i, 0)),
            pl.BlockSpec((1, _BN, 80), lambda i: (0, i, 0)),
            pl.BlockSpec((1, _BN, 80), lambda i: (1, i, 0)),
            pl.BlockSpec((_BN, 1), lambda i: (i, 0)),
        ],
        out_specs=pl.BlockSpec((_BN, 128), lambda i: (i, 0)),
        out_shape=jax.ShapeDtypeStruct((N_NODES, 128), jnp.float32),
    )(sc, agg, agg, node_deg)


def _final_body(sc_ref, agga_ref, aggb_ref, deg_ref, batch_ref, wc_ref, bc_ref,
                out_ref, pooled_acc, cnt_acc):
    i = pl.program_id(0)

    @pl.when(i == 0)
    def _():
        pooled_acc[...] = jnp.zeros_like(pooled_acc)
        cnt_acc[...] = jnp.zeros_like(cnt_acc)

    agg = jnp.concatenate([agga_ref[0], aggb_ref[0]], axis=1)
    h = sc_ref[...] + agg / deg_ref[...]
    b = batch_ref[...]
    rows = []
    cnts = []
    for g in range(N_GRAPHS):
        m = (b == g).astype(jnp.float32)
        cnts.append(jnp.sum(m, axis=0, keepdims=True))
        rows.append(jnp.sum(h * m, axis=0, keepdims=True))
    pooled_acc[...] += jnp.concatenate(rows, axis=0)
    cnt_acc[...] += jnp.concatenate(cnts, axis=0)

    @pl.when(i == pl.num_programs(0) - 1)
    def _():
        pooled = pooled_acc[...] / jnp.maximum(cnt_acc[...], 1.0)
        logit = (jnp.dot(pooled, wc_ref[...], preferred_element_type=jnp.float32)
                 + bc_ref[...])
        out_ref[...] = jax.nn.sigmoid(logit)


def _final(sc, agg, node_deg, batch, Wc, bc):
    full = lambda shape: pl.BlockSpec(shape, lambda i: tuple(0 for _ in shape))
    return pl.pallas_call(
        _final_body,
        grid=(N_NODES // _BN,),
        in_specs=[
            pl.BlockSpec((_BN, 64), lambda i: (i, 0)),
            pl.BlockSpec((1, _BN, 32), lambda i: (0, i, 0)),
            pl.BlockSpec((1, _BN, 32), lambda i: (1, i, 0)),
            pl.BlockSpec((_BN, 1), lambda i: (i, 0)),
            pl.BlockSpec((_BN, 1), lambda i: (i, 0)),
            full((64, 1)),
            full((1, 1)),
        ],
        out_specs=full((N_GRAPHS, 1)),
        out_shape=jax.ShapeDtypeStruct((N_GRAPHS, 1), jnp.float32),
        scratch_shapes=[
            pltpu.VMEM((N_GRAPHS, 64), jnp.float32),
            pltpu.VMEM((N_GRAPHS, 1), jnp.float32),
        ],
    )(sc, agg, agg, node_deg, batch.reshape(N_NODES, 1), Wc, bc.reshape(1, 1))


def _sc_edge(hm3, src, dst, rad3, C2, CR):
    e_per_t = N_EDGES // _NS
    n_chunks = e_per_t // _K
    rpt = 624
    tail_base = rpt * _NS
    tail_rows = N_NODES - tail_base
    mesh = plsc.VectorSubcoreMesh(core_axis_name="c", subcore_axis_name="s")

    @functools.partial(
        pl.kernel,
        mesh=mesh,
        out_type=jax.ShapeDtypeStruct((_NC, N_NODES, C2), jnp.float32),
        scratch_types=[
            pltpu.VMEM_SHARED((N_NODES, C2), jnp.float32),
            pltpu.VMEM((_NSLOT, _K), jnp.int32),
            pltpu.VMEM((_NSLOT, _K), jnp.int32),
            pltpu.VMEM((_NSLOT, _K, CR), jnp.float32),
            pltpu.VMEM((_NSLOT, _K, C2), jnp.float32),
        ] + [pltpu.SemaphoreType.DMA] * (3 * _NSLOT),
        compiler_params=pltpu.CompilerParams(use_tc_tiling_on_sc=False),
    )
    def k(hm_hbm, src_hbm, dst_hbm, rad_hbm, out_hbm, agg_sh, src_v, dst_v,
          rad_v, rows_v, *sems):
        sem_in = sems[0:_NSLOT]
        sem_g = sems[_NSLOT:2 * _NSLOT]
        sem_s = sems[2 * _NSLOT:3 * _NSLOT]
        cid = lax.axis_index("c")
        sid = lax.axis_index("s")

        zero = jnp.zeros((16,), jnp.float32)

        def zrow(i, _):
            for c in range(C2 // 16):
                rows_v[0, i, pl.ds(c * 16, 16)] = zero
            return 0

        lax.fori_loop(0, 16, zrow, 0)

        def zcopy(i, _):
            pltpu.sync_copy(rows_v.at[0, pl.ds(0, 16)],
                            agg_sh.at[pl.ds(sid * rpt + i * 16, 16)])
            return 0

        lax.fori_loop(0, rpt // 16, zcopy, 0)

        @pl.when(sid == _NS - 1)
        def _():
            pltpu.sync_copy(rows_v.at[0, pl.ds(0, 16)],
                            agg_sh.at[pl.ds(tail_base, tail_rows)])

        plsc.subcore_barrier()

        def in_start(i, b):
            base = sid * e_per_t + i * _K
            pltpu.async_copy(src_hbm.at[pl.ds(base, _K)], src_v.at[b], sem_in[b])
            pltpu.async_copy(dst_hbm.at[pl.ds(base, _K)], dst_v.at[b], sem_in[b])
            pltpu.async_copy(rad_hbm.at[cid, pl.ds(base, _K)], rad_v.at[b],
                             sem_in[b])

        def in_wait(i, b):
            base = sid * e_per_t + i * _K
            pltpu.make_async_copy(src_hbm.at[pl.ds(base, _K)], src_v.at[b],
                                  sem_in[b]).wait()
            pltpu.make_async_copy(dst_hbm.at[pl.ds(base, _K)], dst_v.at[b],
                                  sem_in[b]).wait()
            pltpu.make_async_copy(rad_hbm.at[cid, pl.ds(base, _K)],
                                  rad_v.at[b], sem_in[b]).wait()

        def gather_start(b):
            pltpu.async_copy(hm_hbm.at[cid].at[src_v.at[b]], rows_v.at[b],
                             sem_g[b])

        def gather_wait(b):
            pltpu.make_async_copy(hm_hbm.at[cid].at[src_v.at[b]], rows_v.at[b],
                                  sem_g[b]).wait()

        def scatter_start(b):
            pltpu.async_copy(rows_v.at[b], agg_sh.at[dst_v.at[b]], sem_s[b],
                             add=True)

        def scatter_wait(b):
            pltpu.make_async_copy(rows_v.at[b], agg_sh.at[dst_v.at[b]],
                                  sem_s[b]).wait()

        def mul(b):
            def mrow(j, _):
                for c in range(C2 // 16):
                    sl = pl.ds(c * 16, 16)
                    rows_v[b, j, sl] = rows_v[b, j, sl] * rad_v[b, j, sl]
                return 0

            lax.fori_loop(0, _K, mrow, 0)

        in_start(0, 0)
        in_start(1, 1)
        in_wait(0, 0)
        gather_start(0)

        def body(i, _):
            for b in range(_NSLOT):
                @pl.when(i % _NSLOT == b)
                def _():
                    nb = (b + 1) % _NSLOT
                    pb = (b + 2) % _NSLOT

                    @pl.when(i + 1 < n_chunks)
                    def _():
                        in_wait(i + 1, nb)
                        gather_start(nb)

                    gather_wait(b)
                    mul(b)
                    scatter_start(b)

                    @pl.when(i + 2 < n_chunks)
                    def _():
                        @pl.when(i >= 1)
                        def _():
                            scatter_wait(pb)
                        in_start(i + 2, pb)
            return 0

        lax.fori_loop(0, n_chunks, body, 0)

        for j in range(_NSLOT):
            i = n_chunks - _NSLOT + j
            scatter_wait(i % _NSLOT)

        plsc.subcore_barrier()
        pltpu.sync_copy(agg_sh.at[pl.ds(sid * rpt, rpt)],
                        out_hbm.at[cid, pl.ds(sid * rpt, rpt)])

        @pl.when(sid == _NS - 1)
        def _():
            pltpu.sync_copy(agg_sh.at[pl.ds(tail_base, tail_rows)],
                            out_hbm.at[cid, pl.ds(tail_base, tail_rows)])

    return k(hm3, src, dst, rad3)


def kernel(x, z, edge_index, edge_vec, edge_len, r_max, node_deg, numb, batch,
           Wemx, bemx, Wemz, bemz,
           Ws1, Wz1, Wm1, Wra1, bra1, Wrb1, wsh1,
           Ws2, Wz2, Wm2, Wra2, bra2, Wrb2, wsh2,
           Ws3, Wz3, Wm3, Wra3, bra3, Wrb3, wsh3,
           Wc, bc):
    src = edge_index[0]
    dst = edge_index[1]

    h, z0 = _embed(x, z, Wemx, bemx, Wemz, bemz)

    rad1, rad2, rad3 = _radials(edge_len, edge_vec, r_max,
                                Wra1, bra1, Wrb1, wsh1,
                                Wra2, bra2, Wrb2, wsh2,
                                Wra3, bra3, Wrb3, wsh3)

    sc1, hm1 = _pre(h, z0, Ws1, Wz1, Wm1)
    agg1 = _sc_edge(hm1, src, dst, rad1, 80, 80)
    h = _gate(sc1, agg1, node_deg)

    sc2, hm2 = _pre(h, z0, Ws2, Wz2, Wm2)
    agg2 = _sc_edge(hm2, src, dst, rad2, 80, 80)
    h = _gate(sc2, agg2, node_deg)

    sc3, hm3 = _pre(h, z0, Ws3, Wz3, Wm3)
    agg3 = _sc_edge(hm3, src, dst, rad3, 32, 32)
    return _final(sc3, agg3, node_deg, batch, Wc, bc)
